# baseline (speedup 1.0000x reference)
import functools

import jax
import jax.numpy as jnp
from jax import lax
from jax.experimental import pallas as pl
from jax.experimental.pallas import tpu as pltpu
from jax.experimental.pallas import tpu_sc as plsc

_NC = 2
_NS = 16
_NW = _NC * _NS
_WC = 3840
_NCH = 26
_REM0 = _NCH * _WC
_TAIL0 = _REM0 + 128
_NTC = 256
_RT = 32
_D = 4


def _sc_partials(pred_hbm, tail_hbm, out_hbm,
                 b0, b1, rembuf, tailbuf, obuf, a1r, a2r, sems, semr, semt,
                 *, Bsc):
    rpw = Bsc // _NW
    ngrp = rpw // 8
    tch = ngrp * _NCH
    wid = lax.axis_index("s") * _NC + lax.axis_index("c")
    row0 = wid * rpw

    bufs = (b0, b1)
    neg = jnp.full((16,), -jnp.inf, jnp.float32)

    def start_main(t, b):
        gr0 = row0 + (t // _NCH) * 8
        col = (t % _NCH) * _WC
        pltpu.make_async_copy(
            pred_hbm.at[pl.ds(gr0, 8), pl.ds(col, _WC)], bufs[b], sems.at[b]
        ).start()

    start_main(0, 0)
    start_main(1, 1)

    def top2_fold(buf, r, a1, a2, ngroups):
        for h in range(ngroups):
            v = buf[r, pl.ds(h * 16, 16)]
            a2 = jnp.maximum(a2, jnp.minimum(a1, v))
            a1 = jnp.maximum(a1, v)
        return a1, a2

    @pl.loop(0, tch, step=2)
    def _chunks(g):
        for b in range(2):
            t = g + b
            buf = bufs[b]
            pltpu.make_async_copy(
                pred_hbm.at[pl.ds(row0, 8), pl.ds(0, _WC)], buf, sems.at[b]
            ).wait()
            rg = t // _NCH
            pos = t - rg * _NCH
            gr0 = row0 + rg * 8

            @pl.when(pos == 0)
            def _prime_group():
                pltpu.make_async_copy(
                    pred_hbm.at[pl.ds(gr0, 8), pl.ds(_REM0, 128)],
                    rembuf, semr).start()
                pltpu.make_async_copy(
                    tail_hbm.at[pl.ds(gr0, 8)], tailbuf, semt).start()
                for i in range(8):
                    a1r[pl.ds(16 * i, 16)] = neg
                    a2r[pl.ds(16 * i, 16)] = neg

            carry = []
            for r in range(8):
                carry += [a1r[pl.ds(16 * r, 16)], a2r[pl.ds(16 * r, 16)]]
            carry = tuple(carry)

            def mb(tt, carry):
                out = []
                for r in range(8):
                    a1, a2 = carry[2 * r], carry[2 * r + 1]
                    for h in range(8):
                        v = buf[r, pl.ds(tt * 128 + h * 16, 16)]
                        a2 = jnp.maximum(a2, jnp.minimum(a1, v))
                        a1 = jnp.maximum(a1, v)
                    out += [a1, a2]
                return tuple(out)

            carry = lax.fori_loop(0, _WC // 128, mb, carry)
            for r in range(8):
                a1r[pl.ds(16 * r, 16)] = carry[2 * r]
                a2r[pl.ds(16 * r, 16)] = carry[2 * r + 1]

            @pl.when(t + 2 < tch)
            def _prefetch():
                start_main(t + 2, b)

            @pl.when(pos == _NCH - 1)
            def _finalize():
                pltpu.make_async_copy(
                    pred_hbm.at[pl.ds(row0, 8), pl.ds(0, 128)],
                    rembuf, semr).wait()
                pltpu.make_async_copy(
                    pred_hbm.at[pl.ds(row0, 8), pl.ds(0, 128)],
                    tailbuf, semt).wait()
                for r in range(8):
                    rl = rg * 8 + r
                    a1, a2 = a1r[pl.ds(16 * r, 16)], a2r[pl.ds(16 * r, 16)]
                    a1, a2 = top2_fold(rembuf, r, a1, a2, 8)
                    a1, a2 = top2_fold(tailbuf, r, a1, a2, 8)
                    obuf[pl.ds(rl * 32, 16)] = a1
                    obuf[pl.ds(rl * 32 + 16, 16)] = a2

    pltpu.sync_copy(obuf, out_hbm.at[pl.ds(row0 * 32, rpw * 32)])


def _tc_stream(k_ref, pred_ref, out_ref, *, C):
    i = pl.program_id(0)
    C_al = (C // 128) * 128
    lane = jax.lax.broadcasted_iota(jnp.int32, (1, 128), 1)

    pks = []
    for r in range(_RT):
        c = k_ref[i * _RT + r]
        c0 = (c // 128) * 128
        chunk = pred_ref[pl.ds(r, 1), pl.ds(c0, 128)]
        is_l = lane == (c - c0)
        pks.append(jnp.where(is_l, chunk, -jnp.inf).max(axis=1, keepdims=True))
        pred_ref[pl.ds(r, 1), pl.ds(c0, 128)] = jnp.where(is_l, -jnp.inf, chunk)

    main = pred_ref[:, :C_al]
    m = jnp.max(main, axis=1)
    tail = pred_ref[:, C_al:]
    tmask = jax.lax.broadcasted_iota(jnp.int32, tail.shape, 1) < (C - C_al)
    m = jnp.maximum(m, jnp.where(tmask, tail, -jnp.inf).max(axis=1))

    pk = jnp.concatenate(pks, axis=0)
    out_ref[...] = m[:, None] - pk


def _tc_combine(k_smem, part_ref, k2d_ref, pred_hbm, out_ref, bufs, sems):
    i = pl.program_id(0)
    ni = pl.num_programs(0)

    def start_row_copies(step, s):
        base = _NTC + step * 8
        for r in range(8):
            c0 = (k_smem[step * 8 + r] // 128) * 128
            pltpu.make_async_copy(
                pred_hbm.at[pl.ds(base, 8), pl.ds(c0, 128)],
                bufs.at[s * 8 + r], sems.at[s * 8 + r]).start()

    @pl.when(i == 0)
    def _prologue():
        for st in range(_D - 1):
            start_row_copies(st, st)

    @pl.when(i + (_D - 1) < ni)
    def _prefetch():
        start_row_copies(i + (_D - 1), (i + (_D - 1)) % _D)

    a1 = part_ref[:, :16]
    a2 = part_ref[:, 16:]
    lk = jax.lax.rem(k2d_ref[...], 16)
    oh = jax.lax.broadcasted_iota(jnp.int32, (8, 16), 1) == lk
    ninf = jnp.float32(-jnp.inf)
    m_wo = jnp.where(oh, ninf, a1).max(axis=1)
    a1k = jnp.where(oh, a1, ninf).max(axis=1)
    a2k = jnp.where(oh, a2, ninf).max(axis=1)

    s = i % _D
    lane128 = jax.lax.broadcasted_iota(jnp.int32, (1, 128), 1)
    pks = []
    for r in range(8):
        c0 = (k_smem[i * 8 + r] // 128) * 128
        pltpu.make_async_copy(
            pred_hbm.at[pl.ds(_NTC, 8), pl.ds(0, 128)],
            bufs.at[s * 8 + r], sems.at[s * 8 + r]).wait()
        row = bufs[s * 8 + r, pl.ds(r, 1), :]
        ohc = lane128 == (k_smem[i * 8 + r] - c0)
        pks.append(jnp.where(ohc, row, ninf).max(axis=1))
    pk = jnp.concatenate(pks)

    cls = jnp.where(a1k == pk, a2k, a1k)
    out_ref[...] = (jnp.maximum(m_wo, cls) - pk)[:, None]


def kernel(prediction, k):
    B, C = prediction.shape
    Bsc = B - _NTC
    k2 = k.astype(jnp.int32)
    ksc = k2[_NTC:]
    pred_sc = lax.slice(prediction, (_NTC, 0), (B, C))
    tail = jnp.pad(pred_sc[:, _TAIL0:], ((0, 0), (0, 128 - (C - _TAIL0))),
                   constant_values=-jnp.inf)
    rpw = Bsc // _NW
    C_pad = ((C + 127) // 128) * 128
    mesh = plsc.VectorSubcoreMesh(core_axis_name="c", subcore_axis_name="s")

    partials = pl.kernel(
        functools.partial(_sc_partials, Bsc=Bsc),
        out_type=jax.ShapeDtypeStruct((Bsc * 32,), jnp.float32),
        mesh=mesh,
        scratch_types=[
            pltpu.VMEM((8, _WC), jnp.float32),
            pltpu.VMEM((8, _WC), jnp.float32),
            pltpu.VMEM((8, 128), jnp.float32),
            pltpu.VMEM((8, 128), jnp.float32),
            pltpu.VMEM((rpw * 32,), jnp.float32),
            pltpu.VMEM((128,), jnp.float32),
            pltpu.VMEM((128,), jnp.float32),
            pltpu.SemaphoreType.DMA((2,)),
            pltpu.SemaphoreType.DMA,
            pltpu.SemaphoreType.DMA,
        ],
        compiler_params=pltpu.CompilerParams(use_tc_tiling_on_sc=True),
    )(pred_sc, tail)

    out_tc = pl.pallas_call(
        functools.partial(_tc_stream, C=C),
        grid=(_NTC // _RT,),
        in_specs=[
            pl.BlockSpec(memory_space=pltpu.SMEM),
            pl.BlockSpec((_RT, C_pad), lambda i: (i, 0)),
        ],
        out_specs=pl.BlockSpec((_RT, 1), lambda i: (i, 0)),
        out_shape=jax.ShapeDtypeStruct((_NTC, 1), jnp.float32),
        compiler_params=pltpu.CompilerParams(
            dimension_semantics=("arbitrary",),
        ),
    )(k2[:_NTC], prediction)

    part2d = partials.reshape(Bsc, 32)
    out_sc = pl.pallas_call(
        _tc_combine,
        grid=(Bsc // 8,),
        in_specs=[
            pl.BlockSpec(memory_space=pltpu.SMEM),
            pl.BlockSpec((8, 32), lambda i: (i, 0)),
            pl.BlockSpec((8, 1), lambda i: (i, 0)),
            pl.BlockSpec(memory_space=pltpu.MemorySpace.HBM),
        ],
        out_specs=pl.BlockSpec((8, 1), lambda i: (i, 0)),
        out_shape=jax.ShapeDtypeStruct((Bsc, 1), jnp.float32),
        scratch_shapes=[
            pltpu.VMEM((_D * 8, 8, 128), jnp.float32),
            pltpu.SemaphoreType.DMA((_D * 8,)),
        ],
        compiler_params=pltpu.CompilerParams(
            dimension_semantics=("arbitrary",),
        ),
    )(ksc, part2d, ksc.reshape(Bsc, 1), prediction)

    return jnp.concatenate([out_tc.reshape(_NTC), out_sc.reshape(Bsc)])


# speedup vs baseline: 1.6253x; 1.6253x over previous
import functools

import jax
import jax.numpy as jnp
from jax.experimental import pallas as pl
from jax.experimental.pallas import tpu as pltpu

_RT = 32
_BAND = 256


def _tc_stream(k_ref, pred_ref, out_ref, *, C):
    i = pl.program_id(0)
    C_al = (C // 128) * 128
    lane = jax.lax.broadcasted_iota(jnp.int32, (1, 128), 1)

    pks = []
    for r in range(_RT):
        c = k_ref[i * _RT + r]
        c0 = (c // 128) * 128
        chunk = pred_ref[pl.ds(r, 1), pl.ds(c0, 128)]
        is_l = lane == (c - c0)
        pks.append(jnp.where(is_l, chunk, -jnp.inf).max(axis=1, keepdims=True))
        pred_ref[pl.ds(r, 1), pl.ds(c0, 128)] = jnp.where(is_l, -jnp.inf, chunk)

    main = pred_ref[:, :C_al]
    m = jnp.max(main, axis=1)
    tail = pred_ref[:, C_al:]
    tmask = jax.lax.broadcasted_iota(jnp.int32, tail.shape, 1) < (C - C_al)
    m = jnp.maximum(m, jnp.where(tmask, tail, -jnp.inf).max(axis=1))

    pk = jnp.concatenate(pks, axis=0)
    out_ref[...] = m[:, None] - pk


def kernel(prediction, k):
    B, C = prediction.shape
    k2 = k.astype(jnp.int32)
    C_pad = ((C + 127) // 128) * 128
    outs = []
    for band in range(B // _BAND):
        r0 = band * _BAND
        outs.append(pl.pallas_call(
            functools.partial(_tc_stream, C=C),
            grid=(_BAND // _RT,),
            in_specs=[
                pl.BlockSpec(memory_space=pltpu.SMEM),
                pl.BlockSpec((_RT, C_pad), lambda i, band=band: (band * (_BAND // _RT) + i, 0)),
            ],
            out_specs=pl.BlockSpec((_RT, 1), lambda i: (i, 0)),
            out_shape=jax.ShapeDtypeStruct((_BAND, 1), jnp.float32),
            compiler_params=pltpu.CompilerParams(
                dimension_semantics=("arbitrary",),
            ),
        )(k2[r0:r0 + _BAND], prediction).reshape(_BAND))
    return jnp.concatenate(outs)


# speedup vs baseline: 1.6664x; 1.0253x over previous
import functools

import jax
import jax.numpy as jnp
from jax.experimental import pallas as pl
from jax.experimental.pallas import tpu as pltpu

_RT = 64


def _tc_stream(k_ref, pred_ref, out_ref, *, C):
    i = pl.program_id(0)
    C_al = (C // 128) * 128
    lane = jax.lax.broadcasted_iota(jnp.int32, (1, 128), 1)

    pks = []
    for r in range(_RT):
        c = k_ref[i * _RT + r]
        c0 = (c // 128) * 128
        chunk = pred_ref[pl.ds(r, 1), pl.ds(c0, 128)]
        is_l = lane == (c - c0)
        pks.append(jnp.where(is_l, chunk, -jnp.inf).max(axis=1, keepdims=True))
        pred_ref[pl.ds(r, 1), pl.ds(c0, 128)] = jnp.where(is_l, -jnp.inf, chunk)

    main = pred_ref[:, :C_al]
    m = jnp.max(main, axis=1)
    tail = pred_ref[:, C_al:]
    tmask = jax.lax.broadcasted_iota(jnp.int32, tail.shape, 1) < (C - C_al)
    m = jnp.maximum(m, jnp.where(tmask, tail, -jnp.inf).max(axis=1))

    pk = jnp.concatenate(pks, axis=0)
    out_ref[...] = m[:, None] - pk


def kernel(prediction, k):
    B, C = prediction.shape
    k2 = k.astype(jnp.int32)
    C_pad = ((C + 127) // 128) * 128
    out = pl.pallas_call(
        functools.partial(_tc_stream, C=C),
        grid=(B // _RT,),
        in_specs=[
            pl.BlockSpec(memory_space=pltpu.SMEM),
            pl.BlockSpec((_RT, C_pad), lambda i: (i, 0)),
        ],
        out_specs=pl.BlockSpec((_RT, 1), lambda i: (i, 0)),
        out_shape=jax.ShapeDtypeStruct((B, 1), jnp.float32),
        compiler_params=pltpu.CompilerParams(
            dimension_semantics=("arbitrary",),
        ),
    )(k2, prediction)
    return out.reshape(B)


# speedup vs baseline: 1.6882x; 1.0131x over previous
import functools

import jax
import jax.numpy as jnp
from jax.experimental import pallas as pl
from jax.experimental.pallas import tpu as pltpu

_RT = 32


def _tc_stream(k_ref, pred_ref, out_ref, *, C):
    i = pl.program_id(0)
    C_al = (C // 128) * 128
    lane = jax.lax.broadcasted_iota(jnp.int32, (1, 128), 1)

    pks = []
    for r in range(_RT):
        c = k_ref[i * _RT + r]
        c0 = (c // 128) * 128
        chunk = pred_ref[pl.ds(r, 1), pl.ds(c0, 128)]
        is_l = lane == (c - c0)
        pks.append(jnp.where(is_l, chunk, -jnp.inf).max(axis=1, keepdims=True))
        pred_ref[pl.ds(r, 1), pl.ds(c0, 128)] = jnp.where(is_l, -jnp.inf, chunk)

    main = pred_ref[:, :C_al]
    m = jnp.max(main, axis=1)
    tail = pred_ref[:, C_al:]
    tmask = jax.lax.broadcasted_iota(jnp.int32, tail.shape, 1) < (C - C_al)
    m = jnp.maximum(m, jnp.where(tmask, tail, -jnp.inf).max(axis=1))

    pk = jnp.concatenate(pks, axis=0)
    out_ref[...] = m[:, None] - pk


def kernel(prediction, k):
    B, C = prediction.shape
    k2 = k.astype(jnp.int32)
    C_pad = ((C + 127) // 128) * 128
    out = pl.pallas_call(
        functools.partial(_tc_stream, C=C),
        grid=(B // _RT,),
        in_specs=[
            pl.BlockSpec(memory_space=pltpu.SMEM),
            pl.BlockSpec((_RT, C_pad), lambda i: (i, 0)),
        ],
        out_specs=pl.BlockSpec((_RT, 1), lambda i: (i, 0)),
        out_shape=jax.ShapeDtypeStruct((B, 1), jnp.float32),
        compiler_params=pltpu.CompilerParams(
            dimension_semantics=("arbitrary",),
        ),
    )(k2, prediction)
    return out.reshape(B)


# speedup vs baseline: 1.6885x; 1.0002x over previous
import functools

import jax
import jax.numpy as jnp
from jax.experimental import pallas as pl
from jax.experimental.pallas import tpu as pltpu

_RT = 32


def _tc_stream(k_ref, pred_ref, out_ref, *, C):
    i = pl.program_id(0)
    C_al = (C // 128) * 128
    lane = jax.lax.broadcasted_iota(jnp.int32, (1, 128), 1)

    pks = []
    for r in range(_RT):
        c = k_ref[i * _RT + r]
        c0 = (c // 128) * 128
        chunk = pred_ref[pl.ds(r, 1), pl.ds(c0, 128)]
        is_l = lane == (c - c0)
        pks.append(jnp.where(is_l, chunk, -jnp.inf).max(axis=1, keepdims=True))
        pred_ref[pl.ds(r, 1), pl.ds(c0, 128)] = jnp.where(is_l, -jnp.inf, chunk)

    main = pred_ref[:, :C_al]
    m = jnp.max(main, axis=1)
    tail = pred_ref[:, C_al:]
    tmask = jax.lax.broadcasted_iota(jnp.int32, tail.shape, 1) < (C - C_al)
    m = jnp.maximum(m, jnp.where(tmask, tail, -jnp.inf).max(axis=1))

    pk = jnp.concatenate(pks, axis=0)
    out_ref[...] = m[:, None] - pk


def kernel(prediction, k):
    B, C = prediction.shape
    k2 = k.astype(jnp.int32)
    C_pad = ((C + 127) // 128) * 128
    out = pl.pallas_call(
        functools.partial(_tc_stream, C=C),
        grid=(B // _RT,),
        in_specs=[
            pl.BlockSpec(memory_space=pltpu.SMEM),
            pl.BlockSpec((_RT, C_pad), lambda i: (i, 0)),
        ],
        out_specs=pl.BlockSpec((_RT, 1), lambda i: (i, 0)),
        out_shape=jax.ShapeDtypeStruct((B, 1), jnp.float32),
        compiler_params=pltpu.CompilerParams(
            dimension_semantics=("parallel",),
        ),
    )(k2, prediction)
    return out.reshape(B)
